# baseline (speedup 1.0000x reference)
import jax
import jax.numpy as jnp
from jax import lax
from jax.experimental import pallas as pl
from jax.experimental.pallas import tpu as pltpu
from jax.experimental.pallas import tpu_sc as plsc

_LENGTHS = (4096, 3584, 3072, 2560, 2048, 1536, 1024, 512)
_NSEQ = 8
_NCOL = 5
_MAXLEN = 4096
_QUART = 1024
_HALF = 512
_PAD = -1.0

_BNEG = (0.0, 100.0, 100.0, 10.0, -0.0)
_AINV = (1.0, 1.0 / 200.0, 1.0 / 200.0, 1.0 / 20.0, 1.0 / 255.0)

_OUT_WORDS = _NCOL * _NSEQ * _MAXLEN


def _body(*refs):
    ins = refs[:_NSEQ]
    out = refs[_NSEQ]
    lin = refs[_NSEQ + 1]
    sem = refs[_NSEQ + 2]

    core = lax.axis_index("c")
    sub = lax.axis_index("s")
    wid = sub * 2 + core
    j = wid // 4
    q = wid % 4

    neg1 = jnp.full((16,), _PAD, dtype=jnp.float32)

    def _xform(start, n, av, bv):
        @plsc.parallel_loop(0, n, step=16, unroll=4)
        def _(i):
            x = lin[0, pl.ds(start + i, 16)]
            lin[0, pl.ds(start + i, 16)] = (x + bv) * av

    def _fill(start, n):
        @plsc.parallel_loop(0, n, step=16, unroll=4)
        def _(i):
            lin[0, pl.ds(start + i, 16)] = neg1

    def _consts(c):
        return (jnp.full((16,), _AINV[c], dtype=jnp.float32),
                jnp.full((16,), _BNEG[c], dtype=jnp.float32))

    for j0 in range(_NSEQ):
        length = _LENGTHS[j0]
        nhalves = length // _HALF
        nfull = nhalves // 2
        has_half = nhalves % 2 == 1

        @pl.when(j == j0)
        def _seq_block(j0=j0, length=length, nfull=nfull, has_half=has_half):
            out_base = j0 * _MAXLEN
            in_ref = ins[j0]

            def _stage_in(c, lane0, n):
                return pltpu.async_copy(
                    in_ref.at[pl.ds(c, 1), pl.ds(lane0, n)],
                    lin.at[pl.ds(0, 1), pl.ds(c * _QUART, n)],
                    sem,
                )

            def _store_out(c, lane0):
                return pltpu.async_copy(
                    lin.at[0, pl.ds(c * _QUART, _QUART)],
                    out.at[pl.ds(c * (_NSEQ * _MAXLEN) + out_base + lane0,
                                 _QUART)],
                    sem,
                )

            def _full_quarter():
                lane0 = q * _QUART
                descs = [_stage_in(c, lane0, _QUART) for c in range(_NCOL)]
                for d in descs:
                    d.wait()
                out_descs = [_store_out(0, lane0)]
                for c in range(1, _NCOL):
                    av, bv = _consts(c)
                    _xform(c * _QUART, _QUART, av, bv)
                    out_descs.append(_store_out(c, lane0))
                for d in out_descs:
                    d.wait()

            def _half_quarter():
                lane0 = nfull * _QUART
                descs = [_stage_in(c, lane0, _HALF) for c in range(_NCOL)]
                for d in descs:
                    d.wait()
                out_descs = []
                for c in range(_NCOL):
                    if c > 0:
                        av, bv = _consts(c)
                        _xform(c * _QUART, _HALF, av, bv)
                    _fill(c * _QUART + _HALF, _HALF)
                    out_descs.append(_store_out(c, lane0))
                for d in out_descs:
                    d.wait()

            def _pad_quarter():
                lane0 = q * _QUART
                _fill(0, _QUART)
                descs = [
                    pltpu.async_copy(
                        lin.at[0, pl.ds(0, _QUART)],
                        out.at[pl.ds(c * (_NSEQ * _MAXLEN) + out_base
                                     + lane0, _QUART)],
                        sem,
                    )
                    for c in range(_NCOL)
                ]
                for d in descs:
                    d.wait()

            if nfull == 4:
                _full_quarter()
            else:
                if nfull > 0:
                    pl.when(q < nfull)(_full_quarter)
                if has_half:
                    pl.when(q == nfull)(_half_quarter)
                if nfull + (1 if has_half else 0) < 4:
                    pl.when(q >= nfull + (1 if has_half else 0))(_pad_quarter)


def kernel(seq0, seq1, seq2, seq3, seq4, seq5, seq6, seq7):
    seqs = (seq0, seq1, seq2, seq3, seq4, seq5, seq6, seq7)
    ins = tuple(s.T for s in seqs)

    mesh = plsc.VectorSubcoreMesh(core_axis_name="c", subcore_axis_name="s")
    run = pl.kernel(
        _body,
        out_type=jax.ShapeDtypeStruct((_OUT_WORDS,), jnp.float32),
        mesh=mesh,
        scratch_types=[pltpu.VMEM((1, _NCOL * _QUART), jnp.float32),
                       pltpu.SemaphoreType.DMA],
    )
    flat = run(*ins)
    return jnp.transpose(flat.reshape(_NCOL, _NSEQ, _MAXLEN), (1, 2, 0))


# speedup vs baseline: 1.0134x; 1.0134x over previous
import jax
import jax.numpy as jnp
from jax import lax
from jax.experimental import pallas as pl
from jax.experimental.pallas import tpu as pltpu
from jax.experimental.pallas import tpu_sc as plsc

_LENGTHS = (4096, 3584, 3072, 2560, 2048, 1536, 1024, 512)
_NSEQ = 8
_NCOL = 5
_MAXLEN = 4096
_QUART = 1024
_HALF = 512
_PAD = -1.0

_BNEG = (0.0, 100.0, 100.0, 10.0, -0.0)
_AINV = (1.0, 1.0 / 200.0, 1.0 / 200.0, 1.0 / 20.0, 1.0 / 255.0)

_OUT_WORDS = _NCOL * _NSEQ * _MAXLEN


def _body(*refs):
    ins = refs[:_NSEQ]
    out = refs[_NSEQ]
    lin = refs[_NSEQ + 1]
    sem = refs[_NSEQ + 2]

    core = lax.axis_index("c")
    sub = lax.axis_index("s")
    wid = sub * 2 + core
    j = wid // 4
    q = wid % 4

    neg1 = jnp.full((16,), _PAD, dtype=jnp.float32)

    def _xform(start, n, av, bv):
        @plsc.parallel_loop(0, n, step=16, unroll=4)
        def _(i):
            x = lin[0, pl.ds(start + i, 16)]
            lin[0, pl.ds(start + i, 16)] = (x + bv) * av

    def _fill(start, n):
        @plsc.parallel_loop(0, n, step=16, unroll=4)
        def _(i):
            lin[0, pl.ds(start + i, 16)] = neg1

    def _consts(c):
        return (jnp.full((16,), _AINV[c], dtype=jnp.float32),
                jnp.full((16,), _BNEG[c], dtype=jnp.float32))

    for j0 in range(_NSEQ):
        length = _LENGTHS[j0]
        nhalves = length // _HALF
        nfull = nhalves // 2
        has_half = nhalves % 2 == 1

        @pl.when(j == j0)
        def _seq_block(j0=j0, length=length, nfull=nfull, has_half=has_half):
            out_base = j0 * _MAXLEN
            in_ref = ins[j0]

            def _stage_in(c, lane0, n):
                return pltpu.async_copy(
                    in_ref.at[pl.ds(c, 1), pl.ds(lane0, n)],
                    lin.at[pl.ds(0, 1), pl.ds(c * _QUART, n)],
                    sem,
                )

            def _store_out(c, lane0):
                return pltpu.async_copy(
                    lin.at[0, pl.ds(c * _QUART, _QUART)],
                    out.at[pl.ds(c * (_NSEQ * _MAXLEN) + out_base + lane0,
                                 _QUART)],
                    sem,
                )

            def _full_quarter():
                lane0 = q * _QUART
                descs = [_stage_in(c, lane0, _QUART) for c in range(_NCOL)]
                for d in descs:
                    d.wait()
                for c in range(1, _NCOL):
                    av, bv = _consts(c)
                    _xform(c * _QUART, _QUART, av, bv)
                out_descs = [_store_out(c, lane0) for c in range(_NCOL)]
                for d in out_descs:
                    d.wait()

            def _half_quarter():
                lane0 = nfull * _QUART
                descs = [_stage_in(c, lane0, _HALF) for c in range(_NCOL)]
                for d in descs:
                    d.wait()
                for c in range(_NCOL):
                    if c > 0:
                        av, bv = _consts(c)
                        _xform(c * _QUART, _HALF, av, bv)
                    _fill(c * _QUART + _HALF, _HALF)
                out_descs = [_store_out(c, lane0) for c in range(_NCOL)]
                for d in out_descs:
                    d.wait()

            def _pad_quarter():
                lane0 = q * _QUART
                _fill(0, _QUART)
                descs = [
                    pltpu.async_copy(
                        lin.at[0, pl.ds(0, _QUART)],
                        out.at[pl.ds(c * (_NSEQ * _MAXLEN) + out_base
                                     + lane0, _QUART)],
                        sem,
                    )
                    for c in range(_NCOL)
                ]
                for d in descs:
                    d.wait()

            if nfull == 4:
                _full_quarter()
            else:
                if nfull > 0:
                    pl.when(q < nfull)(_full_quarter)
                if has_half:
                    pl.when(q == nfull)(_half_quarter)
                if nfull + (1 if has_half else 0) < 4:
                    pl.when(q >= nfull + (1 if has_half else 0))(_pad_quarter)


def kernel(seq0, seq1, seq2, seq3, seq4, seq5, seq6, seq7):
    seqs = (seq0, seq1, seq2, seq3, seq4, seq5, seq6, seq7)
    ins = tuple(s.T for s in seqs)

    mesh = plsc.VectorSubcoreMesh(core_axis_name="c", subcore_axis_name="s")
    run = pl.kernel(
        _body,
        out_type=jax.ShapeDtypeStruct((_OUT_WORDS,), jnp.float32),
        mesh=mesh,
        scratch_types=[pltpu.VMEM((1, _NCOL * _QUART), jnp.float32),
                       pltpu.SemaphoreType.DMA],
    )
    flat = run(*ins)
    return jnp.transpose(flat.reshape(_NCOL, _NSEQ, _MAXLEN), (1, 2, 0))
